# baseline (speedup 1.0000x reference)
import functools

import jax
import jax.numpy as jnp
from jax.experimental import pallas as pl
from jax.experimental.pallas import tpu as pltpu

_T = 128


def _gemm_body(t_ref, e_ref, lo_ref, hi_ref, init_ref, xs_ref, w_ref, out_ref):
    s = pl.program_id(0)
    t = t_ref[s]
    lo = lo_ref[s]
    hi = hi_ref[s]
    row = t * _T + jax.lax.broadcasted_iota(jnp.int32, (_T, 1), 0)
    mask = (row >= lo) & (row < hi)
    acc = jnp.dot(xs_ref[...], w_ref[0], preferred_element_type=jnp.float32)
    contrib = jnp.where(mask, acc, 0.0)

    @pl.when(init_ref[s] != 0)
    def _init():
        out_ref[...] = contrib

    @pl.when(init_ref[s] == 0)
    def _accum():
        out_ref[...] += contrib


def kernel(x, sel, w):
    M, K = x.shape
    E, _, N = w.shape
    T = _T
    num_tiles = M // T
    S = num_tiles + E

    oh = (sel[:, None] == jnp.arange(E, dtype=jnp.int32)[None, :]).astype(jnp.int32)
    csum = jnp.cumsum(oh, axis=0)
    rank = jnp.sum(oh * csum, axis=1) - 1
    counts = csum[-1]
    off = jnp.concatenate([jnp.zeros((1,), jnp.int32),
                           jnp.cumsum(counts).astype(jnp.int32)])
    inv = jnp.sum(oh * off[None, :E], axis=1) + rank
    iota = jnp.arange(M, dtype=jnp.int32)
    perm = jnp.zeros((M,), jnp.int32).at[inv].set(iota)

    first_tile = off[:E] // T
    last_tile = (off[1:] - 1) // T
    ntiles = jnp.where(counts > 0, last_tile - first_tile + 1, 0).astype(jnp.int32)
    sstart = jnp.concatenate([jnp.zeros((1,), jnp.int32),
                              jnp.cumsum(ntiles).astype(jnp.int32)])
    s_idx = jnp.arange(S, dtype=jnp.int32)
    e_arr = jnp.searchsorted(sstart[1:], s_idx, side='right').astype(jnp.int32)
    e_arr = jnp.clip(e_arr, 0, E - 1)
    valid = s_idx < sstart[E]
    t_arr = first_tile[e_arr] + (s_idx - sstart[e_arr])
    t_arr = jnp.where(valid, t_arr, num_tiles - 1).astype(jnp.int32)
    lo_arr = jnp.where(valid, jnp.maximum(off[e_arr], t_arr * T), 0).astype(jnp.int32)
    hi_arr = jnp.where(valid, jnp.minimum(off[e_arr + 1], (t_arr + 1) * T), 0).astype(jnp.int32)
    init_arr = jnp.concatenate([jnp.ones((1,), jnp.int32),
                                (t_arr[1:] != t_arr[:-1]).astype(jnp.int32)])

    xs = x[perm]

    grid_spec = pltpu.PrefetchScalarGridSpec(
        num_scalar_prefetch=5,
        grid=(S,),
        in_specs=[
            pl.BlockSpec((T, K), lambda s, t, e, lo, hi, ini: (t[s], 0)),
            pl.BlockSpec((1, K, N), lambda s, t, e, lo, hi, ini: (e[s], 0, 0)),
        ],
        out_specs=pl.BlockSpec((T, N), lambda s, t, e, lo, hi, ini: (t[s], 0)),
    )
    ys = pl.pallas_call(
        _gemm_body,
        grid_spec=grid_spec,
        out_shape=jax.ShapeDtypeStruct((M, N), jnp.float32),
    )(t_arr, e_arr, lo_arr, hi_arr, init_arr, xs, w)

    return ys[inv]


# speedup vs baseline: 1.3829x; 1.3829x over previous
import functools

import jax
import jax.numpy as jnp
from jax.experimental import pallas as pl
from jax.experimental.pallas import tpu as pltpu

_T = 128
_SR = 64
_SC = 128


def _route_body(sel_ref, inv_ref, t_ref, e_ref, lo_ref, hi_ref, init_ref,
                *, E, T, num_tiles):
    R, C = _SR, _SC
    sel2 = sel_ref[...]
    li = jax.lax.broadcasted_iota(jnp.int32, (C, C), 0)
    ci = jax.lax.broadcasted_iota(jnp.int32, (C, C), 1)
    U = (li <= ci).astype(jnp.float32)
    lr = jax.lax.broadcasted_iota(jnp.int32, (R, R), 0)
    cr = jax.lax.broadcasted_iota(jnp.int32, (R, R), 1)
    Ls = (lr > cr).astype(jnp.float32)
    ones_c = jnp.ones((C, 1), jnp.float32)

    inv2 = jnp.zeros((R, C), jnp.int32)
    offs = [jnp.int32(0)]
    for e in range(E):
        sel_is_e = sel2 == e
        m = sel_is_e.astype(jnp.float32)
        pref_in = jnp.dot(m, U, preferred_element_type=jnp.float32)
        rowtot = jnp.dot(m, ones_c, preferred_element_type=jnp.float32)
        rowpre = jnp.dot(Ls, rowtot, preferred_element_type=jnp.float32)
        rank = (pref_in - m + rowpre).astype(jnp.int32)
        inv2 = inv2 + jnp.where(sel_is_e, offs[e] + rank, 0)
        offs.append(offs[e] + jnp.sum(m).astype(jnp.int32))
    inv_ref[...] = inv2

    lane = jax.lax.broadcasted_iota(jnp.int32, (1, C), 1)
    t_v = jnp.full((1, C), num_tiles - 1, jnp.int32)
    e_v = jnp.zeros((1, C), jnp.int32)
    lo_v = jnp.zeros((1, C), jnp.int32)
    hi_v = jnp.zeros((1, C), jnp.int32)
    sstart = jnp.int32(0)
    for e in range(E):
        cnt = offs[e + 1] - offs[e]
        ft = offs[e] // T
        lt = (offs[e + 1] - 1) // T
        nt = jnp.where(cnt > 0, lt - ft + 1, 0)
        mask = (lane >= sstart) & (lane < sstart + nt)
        tt = ft + (lane - sstart)
        t_v = jnp.where(mask, tt, t_v)
        e_v = jnp.where(mask, e, e_v)
        lo_v = jnp.where(mask, jnp.maximum(offs[e], tt * T), lo_v)
        hi_v = jnp.where(mask, jnp.minimum(offs[e + 1], (tt + 1) * T), hi_v)
        sstart = sstart + nt
    tshift = pltpu.roll(t_v, 1, axis=1)
    init_v = ((t_v != tshift) | (lane == 0)).astype(jnp.int32)

    t_ref[...] = jnp.broadcast_to(t_v, (8, C))
    e_ref[...] = jnp.broadcast_to(e_v, (8, C))
    lo_ref[...] = jnp.broadcast_to(lo_v, (8, C))
    hi_ref[...] = jnp.broadcast_to(hi_v, (8, C))
    init_ref[...] = jnp.broadcast_to(init_v, (8, C))


def _gemm_body(t_ref, e_ref, lo_ref, hi_ref, init_ref, xs_ref, w_ref, out_ref):
    s = pl.program_id(0)
    t = t_ref[0, s]
    lo = lo_ref[0, s]
    hi = hi_ref[0, s]
    row = t * _T + jax.lax.broadcasted_iota(jnp.int32, (_T, 1), 0)
    mask = (row >= lo) & (row < hi)
    acc = jnp.dot(xs_ref[...], w_ref[0], preferred_element_type=jnp.float32)
    contrib = jnp.where(mask, acc, 0.0)

    @pl.when(init_ref[0, s] != 0)
    def _init():
        out_ref[...] = contrib

    @pl.when(init_ref[0, s] == 0)
    def _accum():
        out_ref[...] += contrib


def kernel(x, sel, w):
    M, K = x.shape
    E, _, N = w.shape
    T = _T
    num_tiles = M // T
    S = num_tiles + E

    i32_8x = jax.ShapeDtypeStruct((8, _SC), jnp.int32)
    inv2, t8, e8, lo8, hi8, init8 = pl.pallas_call(
        functools.partial(_route_body, E=E, T=T, num_tiles=num_tiles),
        out_shape=[jax.ShapeDtypeStruct((_SR, _SC), jnp.int32),
                   i32_8x, i32_8x, i32_8x, i32_8x, i32_8x],
    )(sel.reshape(_SR, _SC))
    inv = inv2.reshape(M)

    perm = jnp.zeros((M,), jnp.int32).at[inv].set(jnp.arange(M, dtype=jnp.int32))
    xs = x[perm]

    grid_spec = pltpu.PrefetchScalarGridSpec(
        num_scalar_prefetch=5,
        grid=(S,),
        in_specs=[
            pl.BlockSpec((T, K), lambda s, t, e, lo, hi, ini: (t[0, s], 0)),
            pl.BlockSpec((1, K, N), lambda s, t, e, lo, hi, ini: (e[0, s], 0, 0)),
        ],
        out_specs=pl.BlockSpec((T, N), lambda s, t, e, lo, hi, ini: (t[0, s], 0)),
    )
    ys = pl.pallas_call(
        _gemm_body,
        grid_spec=grid_spec,
        out_shape=jax.ShapeDtypeStruct((M, N), jnp.float32),
    )(t8, e8, lo8, hi8, init8, xs, w)

    return ys[inv]
